# baseline (speedup 1.0000x reference)
import functools
import math

import jax
import jax.numpy as jnp
from jax import lax
from jax.experimental import pallas as pl
from jax.experimental.pallas import tpu as pltpu
from jax.experimental.pallas import tpu_sc as plsc

ROWS = 2048
SIZE = 32000
PADDING_IDX = 0
SMOOTHING = 0.1
CONFIDENCE = 1.0 - SMOOTHING
EPS = SMOOTHING / (SIZE - 1)
C_CONST = (SIZE - 1) * EPS * math.log(EPS) + CONFIDENCE * math.log(CONFIDENCE)

L = 16
NW = 32

R_SC = 1024
R_TC = ROWS - R_SC
BPWD = R_SC // NW
PW = 3200
NPB = SIZE // PW
NBW = BPWD // 8


RB = 128
CB = SIZE
NI = R_TC // RB


def _tc_body(t_ref, x_ref, oa_ref, ob_ref, ot_ref, acc_ref):
    i = pl.program_id(0)

    @pl.when(i == 0)
    def _init():
        acc_ref[0] = 0.0
        acc_ref[1] = 0.0
        acc_ref[2] = 0.0

    t = t_ref[0]
    mcol = t != PADDING_IDX
    mf = mcol.astype(jnp.float32)
    xb = x_ref[...]
    colid = lax.broadcasted_iota(jnp.int32, (RB, CB), 1)
    acc_ref[0] += jnp.sum(xb * mf)
    acc_ref[1] += jnp.sum(jnp.where((colid == t) & mcol, xb, 0.0))
    acc_ref[2] += jnp.sum(mf)

    @pl.when(i == NI - 1)
    def _fin():
        oa_ref[0, 0] = acc_ref[0]
        ob_ref[0, 0] = acc_ref[1]
        ot_ref[0, 0] = acc_ref[2]


def _tc_part(x, target):
    t3 = target[:R_TC].reshape(NI, RB, 1)
    scalar_spec = pl.BlockSpec((1, 1), lambda i: (0, 0),
                               memory_space=pltpu.SMEM)
    outs = pl.pallas_call(
        _tc_body,
        grid=(NI,),
        in_specs=[
            pl.BlockSpec((1, RB, 1), lambda i: (i, 0, 0)),
            pl.BlockSpec((RB, CB), lambda i: (i, 0)),
        ],
        out_specs=[scalar_spec, scalar_spec, scalar_spec],
        out_shape=[jax.ShapeDtypeStruct((1, 1), jnp.float32)] * 3,
        scratch_shapes=[pltpu.SMEM((3,), jnp.float32)],
    )(t3, x)
    return outs[0][0, 0], outs[1][0, 0], outs[2][0, 0]


def _sc_kernel(x_hbm, texp_hbm, mexp_hbm, a_hbm, b_hbm, tok_hbm,
               texp_v, mexp_v, out_v, band0, band1, sem0, sem1):
    wid = lax.axis_index("s") * 2 + lax.axis_index("c")
    vzero = jnp.zeros((L,), jnp.float32)
    lanes = lax.iota(jnp.int32, L)
    dbase = R_TC + wid * BPWD
    pltpu.sync_copy(texp_hbm.at[pl.ds(wid * BPWD * L, BPWD * L)], texp_v)
    pltpu.sync_copy(mexp_hbm.at[pl.ds(wid * BPWD * L, BPWD * L)], mexp_v)

    bufs = (band0, band1)
    sems = (sem0, sem1)

    def start(i, p, slot):
        return pltpu.async_copy(
            x_hbm.at[pl.ds(dbase + i * 8, 8), pl.ds(p * PW, PW)],
            bufs[slot], sems[slot])

    UNROLL = 8

    def band_loop(i, carry):
        aacc, bacc, tacc = carry
        asr = [vzero] * 8
        gsr = [vzero] * 8
        for p in range(NPB):
            slot = p % 2
            if p + 1 < NPB:
                start(i, p + 1, (p + 1) % 2)
            else:
                @pl.when(i + 1 < NBW)
                def _prefetch():
                    start(i + 1, 0, 0)
            pltpu.make_async_copy(
                x_hbm.at[pl.ds(dbase, 8), pl.ds(0, PW)],
                bufs[slot], sems[slot]).wait()
            buf = bufs[slot]
            for sr in range(8):
                t16 = texp_v[pl.ds((i * 8 + sr) * L, L)]
                tlp = t16 - lanes - p * PW

                def it(j, carry2, buf=buf, sr=sr, tlp=tlp):
                    a, g = carry2
                    tlj = tlp - j * (UNROLL * L)
                    for u in range(UNROLL):
                        c = buf[sr, pl.ds(j * (UNROLL * L) + u * L, L)]
                        a = a + c
                        g = g + jnp.where(tlj == u * L, c, 0.0)
                    return (a, g)
                asr[sr], gsr[sr] = lax.fori_loop(0, PW // (UNROLL * L), it,
                                                 (asr[sr], gsr[sr]))
        for sr in range(8):
            mrow = mexp_v[pl.ds((i * 8 + sr) * L, L)]
            aacc = aacc + asr[sr] * mrow
            bacc = bacc + gsr[sr] * mrow
            tacc = tacc + mrow
        return (aacc, bacc, tacc)

    start(0, 0, 0)
    aacc, bacc, tacc = lax.fori_loop(0, NBW, band_loop, (vzero,) * 3)
    inv_l = jnp.full((L,), 1.0 / L, jnp.float32)
    out_v[...] = aacc
    pltpu.sync_copy(out_v, a_hbm.at[wid])
    out_v[...] = bacc
    pltpu.sync_copy(out_v, b_hbm.at[wid])
    out_v[...] = tacc * inv_l
    pltpu.sync_copy(out_v, tok_hbm.at[wid])


@functools.cache
def _make_sc_call():
    return functools.partial(
        pl.kernel,
        mesh=plsc.VectorSubcoreMesh(core_axis_name="c", subcore_axis_name="s"),
        out_type=[
            jax.ShapeDtypeStruct((NW, L), jnp.float32),
            jax.ShapeDtypeStruct((NW, L), jnp.float32),
            jax.ShapeDtypeStruct((NW, L), jnp.float32),
        ],
        scratch_types=[
            pltpu.VMEM((BPWD * L,), jnp.int32),
            pltpu.VMEM((BPWD * L,), jnp.float32),
            pltpu.VMEM((L,), jnp.float32),
            pltpu.VMEM((8, PW), jnp.float32),
            pltpu.VMEM((8, PW), jnp.float32),
            pltpu.SemaphoreType.DMA,
            pltpu.SemaphoreType.DMA,
        ],
    )(_sc_kernel)


def kernel(x, target):
    target = target.astype(jnp.int32)
    tsc = target[R_TC:]
    texp = jnp.broadcast_to(tsc[:, None], (R_SC, L)).reshape(-1)
    mexp = jnp.broadcast_to(
        (tsc != PADDING_IDX).astype(jnp.float32)[:, None],
        (R_SC, L)).reshape(-1)
    a_parts, b_parts, tok_parts = _make_sc_call()(x, texp, mexp)
    a_tc, b_tc, tok_tc = _tc_part(x, target)
    a_sum = a_tc + jnp.sum(a_parts)
    b_sum = b_tc + jnp.sum(b_parts)
    tokens = tok_tc + jnp.sum(tok_parts)
    c32 = jnp.float32(C_CONST)
    return c32 - (jnp.float32(EPS) * a_sum
                  + jnp.float32(CONFIDENCE - EPS) * b_sum) / tokens


# speedup vs baseline: 1.0374x; 1.0374x over previous
import functools
import math

import jax
import jax.numpy as jnp
from jax import lax
from jax.experimental import pallas as pl
from jax.experimental.pallas import tpu as pltpu
from jax.experimental.pallas import tpu_sc as plsc

ROWS = 2048
SIZE = 32000
PADDING_IDX = 0
SMOOTHING = 0.1
CONFIDENCE = 1.0 - SMOOTHING
EPS = SMOOTHING / (SIZE - 1)
C_CONST = (SIZE - 1) * EPS * math.log(EPS) + CONFIDENCE * math.log(CONFIDENCE)

L = 16
NW = 32

R_SC = 960
R_TC = ROWS - R_SC
BPWD = R_SC // NW


RB = 136
CB = SIZE
NI = R_TC // RB


def _tc_body(t_ref, x_ref, oa_ref, ob_ref, ot_ref, acc_ref):
    i = pl.program_id(0)

    @pl.when(i == 0)
    def _init():
        acc_ref[0] = 0.0
        acc_ref[1] = 0.0
        acc_ref[2] = 0.0

    t = t_ref[0]
    mcol = t != PADDING_IDX
    mf = mcol.astype(jnp.float32)
    xb = x_ref[...]
    colid = lax.broadcasted_iota(jnp.int32, (RB, CB), 1)
    acc_ref[0] += jnp.sum(xb * mf)
    acc_ref[1] += jnp.sum(jnp.where((colid == t) & mcol, xb, 0.0))
    acc_ref[2] += jnp.sum(mf)

    @pl.when(i == NI - 1)
    def _fin():
        oa_ref[0, 0] = acc_ref[0]
        ob_ref[0, 0] = acc_ref[1]
        ot_ref[0, 0] = acc_ref[2]


def _tc_part(x, target):
    t3 = target[:R_TC].reshape(NI, RB, 1)
    scalar_spec = pl.BlockSpec((1, 1), lambda i: (0, 0),
                               memory_space=pltpu.SMEM)
    outs = pl.pallas_call(
        _tc_body,
        grid=(NI,),
        in_specs=[
            pl.BlockSpec((1, RB, 1), lambda i: (i, 0, 0)),
            pl.BlockSpec((RB, CB), lambda i: (i, 0)),
        ],
        out_specs=[scalar_spec, scalar_spec, scalar_spec],
        out_shape=[jax.ShapeDtypeStruct((1, 1), jnp.float32)] * 3,
        scratch_shapes=[pltpu.SMEM((3,), jnp.float32)],
    )(t3, x)
    return outs[0][0, 0], outs[1][0, 0], outs[2][0, 0]


def _sc_kernel(x_hbm, texp_hbm, mexp_hbm, a_hbm, b_hbm, tok_hbm,
               texp_v, mexp_v, out_v, row0, row1, sem0, sem1):
    wid = lax.axis_index("s") * 2 + lax.axis_index("c")
    vzero = jnp.zeros((L,), jnp.float32)
    lanes = lax.iota(jnp.int32, L)
    dbase = R_TC + wid * BPWD
    pltpu.sync_copy(texp_hbm.at[pl.ds(wid * BPWD * L, BPWD * L)], texp_v)
    pltpu.sync_copy(mexp_hbm.at[pl.ds(wid * BPWD * L, BPWD * L)], mexp_v)

    UNROLL = 16

    def row_scan(buf, t16):
        tl = t16 - lanes

        def it(k, accs):
            a0, a1, a2, a3, g = accs
            o = k * (UNROLL * L)
            tlk = tl - o
            aa = [a0, a1, a2, a3]
            for j in range(UNROLL):
                c = buf[pl.ds(o + j * L, L)]
                aa[j % 4] = aa[j % 4] + c
                g = g + jnp.where(tlk == j * L, c, 0.0)
            return (aa[0], aa[1], aa[2], aa[3], g)
        a0, a1, a2, a3, g = lax.fori_loop(0, SIZE // (UNROLL * L), it,
                                          (vzero,) * 5)
        return (a0 + a1) + (a2 + a3), g

    bufs = (row0, row1)
    sems = (sem0, sem1)
    cps = [None, None]
    cps[0] = pltpu.async_copy(x_hbm.at[dbase], row0, sem0)
    aacc = vzero
    bacc = vzero
    tacc = vzero
    for r in range(BPWD):
        b = r % 2
        if r + 1 < BPWD:
            nb = (r + 1) % 2
            cps[nb] = pltpu.async_copy(
                x_hbm.at[dbase + r + 1], bufs[nb], sems[nb])
        cps[b].wait()
        t16 = texp_v[pl.ds(r * L, L)]
        mrow = mexp_v[pl.ds(r * L, L)]
        rowvec, g16 = row_scan(bufs[b], t16)
        aacc = aacc + rowvec * mrow
        bacc = bacc + g16 * mrow
        tacc = tacc + mrow
    inv_l = jnp.full((L,), 1.0 / L, jnp.float32)
    out_v[...] = aacc
    pltpu.sync_copy(out_v, a_hbm.at[wid])
    out_v[...] = bacc
    pltpu.sync_copy(out_v, b_hbm.at[wid])
    out_v[...] = tacc * inv_l
    pltpu.sync_copy(out_v, tok_hbm.at[wid])


@functools.cache
def _make_sc_call():
    return functools.partial(
        pl.kernel,
        mesh=plsc.VectorSubcoreMesh(core_axis_name="c", subcore_axis_name="s"),
        out_type=[
            jax.ShapeDtypeStruct((NW, L), jnp.float32),
            jax.ShapeDtypeStruct((NW, L), jnp.float32),
            jax.ShapeDtypeStruct((NW, L), jnp.float32),
        ],
        scratch_types=[
            pltpu.VMEM((BPWD * L,), jnp.int32),
            pltpu.VMEM((BPWD * L,), jnp.float32),
            pltpu.VMEM((L,), jnp.float32),
            pltpu.VMEM((SIZE,), jnp.float32),
            pltpu.VMEM((SIZE,), jnp.float32),
            pltpu.SemaphoreType.DMA,
            pltpu.SemaphoreType.DMA,
        ],
    )(_sc_kernel)


def kernel(x, target):
    target = target.astype(jnp.int32)
    tsc = target[R_TC:]
    texp = jnp.broadcast_to(tsc[:, None], (R_SC, L)).reshape(-1)
    mexp = jnp.broadcast_to(
        (tsc != PADDING_IDX).astype(jnp.float32)[:, None],
        (R_SC, L)).reshape(-1)
    a_parts, b_parts, tok_parts = _make_sc_call()(x, texp, mexp)
    a_tc, b_tc, tok_tc = _tc_part(x, target)
    a_sum = a_tc + jnp.sum(a_parts)
    b_sum = b_tc + jnp.sum(b_parts)
    tokens = tok_tc + jnp.sum(tok_parts)
    c32 = jnp.float32(C_CONST)
    return c32 - (jnp.float32(EPS) * a_sum
                  + jnp.float32(CONFIDENCE - EPS) * b_sum) / tokens


# speedup vs baseline: 1.0556x; 1.0175x over previous
import functools
import math

import jax
import jax.numpy as jnp
from jax import lax
from jax.experimental import pallas as pl
from jax.experimental.pallas import tpu as pltpu
from jax.experimental.pallas import tpu_sc as plsc

ROWS = 2048
SIZE = 32000
PADDING_IDX = 0
SMOOTHING = 0.1
CONFIDENCE = 1.0 - SMOOTHING
EPS = SMOOTHING / (SIZE - 1)
C_CONST = (SIZE - 1) * EPS * math.log(EPS) + CONFIDENCE * math.log(CONFIDENCE)

L = 16
NW = 32

R_SC = 896
R_TC = ROWS - R_SC
BPWD = R_SC // NW


RB = 128
CB = SIZE
NI = R_TC // RB


def _tc_body(t_ref, x_ref, oa_ref, ob_ref, ot_ref, acc_ref):
    i = pl.program_id(0)

    @pl.when(i == 0)
    def _init():
        acc_ref[0] = 0.0
        acc_ref[1] = 0.0
        acc_ref[2] = 0.0

    t = t_ref[0]
    mcol = t != PADDING_IDX
    mf = mcol.astype(jnp.float32)
    xb = x_ref[...]
    colid = lax.broadcasted_iota(jnp.int32, (RB, CB), 1)
    acc_ref[0] += jnp.sum(xb * mf)
    acc_ref[1] += jnp.sum(jnp.where((colid == t) & mcol, xb, 0.0))
    acc_ref[2] += jnp.sum(mf)

    @pl.when(i == NI - 1)
    def _fin():
        oa_ref[0, 0] = acc_ref[0]
        ob_ref[0, 0] = acc_ref[1]
        ot_ref[0, 0] = acc_ref[2]


def _tc_part(x, target):
    t3 = target[:R_TC].reshape(NI, RB, 1)
    scalar_spec = pl.BlockSpec((1, 1), lambda i: (0, 0),
                               memory_space=pltpu.SMEM)
    outs = pl.pallas_call(
        _tc_body,
        grid=(NI,),
        in_specs=[
            pl.BlockSpec((1, RB, 1), lambda i: (i, 0, 0)),
            pl.BlockSpec((RB, CB), lambda i: (i, 0)),
        ],
        out_specs=[scalar_spec, scalar_spec, scalar_spec],
        out_shape=[jax.ShapeDtypeStruct((1, 1), jnp.float32)] * 3,
        scratch_shapes=[pltpu.SMEM((3,), jnp.float32)],
    )(t3, x)
    return outs[0][0, 0], outs[1][0, 0], outs[2][0, 0]


def _sc_kernel(x_hbm, texp_hbm, mexp_hbm, a_hbm, b_hbm, tok_hbm,
               texp_v, mexp_v, out_v, row0, row1, sem0, sem1):
    wid = lax.axis_index("s") * 2 + lax.axis_index("c")
    vzero = jnp.zeros((L,), jnp.float32)
    lanes = lax.iota(jnp.int32, L)
    dbase = R_TC + wid * BPWD
    pltpu.sync_copy(texp_hbm.at[pl.ds(wid * BPWD * L, BPWD * L)], texp_v)
    pltpu.sync_copy(mexp_hbm.at[pl.ds(wid * BPWD * L, BPWD * L)], mexp_v)

    UNROLL = 16

    def row_scan(buf, t16):
        tl = t16 - lanes

        def it(k, accs):
            a0, a1, a2, a3, g = accs
            o = k * (UNROLL * L)
            tlk = tl - o
            aa = [a0, a1, a2, a3]
            for j in range(UNROLL):
                c = buf[pl.ds(o + j * L, L)]
                aa[j % 4] = aa[j % 4] + c
                g = g + jnp.where(tlk == j * L, c, 0.0)
            return (aa[0], aa[1], aa[2], aa[3], g)
        a0, a1, a2, a3, g = lax.fori_loop(0, SIZE // (UNROLL * L), it,
                                          (vzero,) * 5)
        return (a0 + a1) + (a2 + a3), g

    bufs = (row0, row1)
    sems = (sem0, sem1)
    cps = [None, None]
    cps[0] = pltpu.async_copy(x_hbm.at[dbase], row0, sem0)
    aacc = vzero
    bacc = vzero
    tacc = vzero
    for r in range(BPWD):
        b = r % 2
        if r + 1 < BPWD:
            nb = (r + 1) % 2
            cps[nb] = pltpu.async_copy(
                x_hbm.at[dbase + r + 1], bufs[nb], sems[nb])
        cps[b].wait()
        t16 = texp_v[pl.ds(r * L, L)]
        mrow = mexp_v[pl.ds(r * L, L)]
        rowvec, g16 = row_scan(bufs[b], t16)
        aacc = aacc + rowvec * mrow
        bacc = bacc + g16 * mrow
        tacc = tacc + mrow
    inv_l = jnp.full((L,), 1.0 / L, jnp.float32)
    out_v[...] = aacc
    pltpu.sync_copy(out_v, a_hbm.at[wid])
    out_v[...] = bacc
    pltpu.sync_copy(out_v, b_hbm.at[wid])
    out_v[...] = tacc * inv_l
    pltpu.sync_copy(out_v, tok_hbm.at[wid])


@functools.cache
def _make_sc_call():
    return functools.partial(
        pl.kernel,
        mesh=plsc.VectorSubcoreMesh(core_axis_name="c", subcore_axis_name="s"),
        out_type=[
            jax.ShapeDtypeStruct((NW, L), jnp.float32),
            jax.ShapeDtypeStruct((NW, L), jnp.float32),
            jax.ShapeDtypeStruct((NW, L), jnp.float32),
        ],
        scratch_types=[
            pltpu.VMEM((BPWD * L,), jnp.int32),
            pltpu.VMEM((BPWD * L,), jnp.float32),
            pltpu.VMEM((L,), jnp.float32),
            pltpu.VMEM((SIZE,), jnp.float32),
            pltpu.VMEM((SIZE,), jnp.float32),
            pltpu.SemaphoreType.DMA,
            pltpu.SemaphoreType.DMA,
        ],
    )(_sc_kernel)


def kernel(x, target):
    target = target.astype(jnp.int32)
    tsc = target[R_TC:]
    texp = jnp.broadcast_to(tsc[:, None], (R_SC, L)).reshape(-1)
    mexp = jnp.broadcast_to(
        (tsc != PADDING_IDX).astype(jnp.float32)[:, None],
        (R_SC, L)).reshape(-1)
    a_parts, b_parts, tok_parts = _make_sc_call()(x, texp, mexp)
    a_tc, b_tc, tok_tc = _tc_part(x, target)
    a_sum = a_tc + jnp.sum(a_parts)
    b_sum = b_tc + jnp.sum(b_parts)
    tokens = tok_tc + jnp.sum(tok_parts)
    c32 = jnp.float32(C_CONST)
    return c32 - (jnp.float32(EPS) * a_sum
                  + jnp.float32(CONFIDENCE - EPS) * b_sum) / tokens


# speedup vs baseline: 1.0632x; 1.0072x over previous
import functools
import math

import jax
import jax.numpy as jnp
from jax import lax
from jax.experimental import pallas as pl
from jax.experimental.pallas import tpu as pltpu
from jax.experimental.pallas import tpu_sc as plsc

ROWS = 2048
SIZE = 32000
PADDING_IDX = 0
SMOOTHING = 0.1
CONFIDENCE = 1.0 - SMOOTHING
EPS = SMOOTHING / (SIZE - 1)
C_CONST = (SIZE - 1) * EPS * math.log(EPS) + CONFIDENCE * math.log(CONFIDENCE)

L = 16
NW = 32

R_SC = 928
R_TC = ROWS - R_SC
BPWD = R_SC // NW


RB = 112
CB = SIZE
NI = R_TC // RB


def _tc_body(t_ref, x_ref, oa_ref, ob_ref, ot_ref, acc_ref):
    i = pl.program_id(0)

    @pl.when(i == 0)
    def _init():
        acc_ref[0] = 0.0
        acc_ref[1] = 0.0
        acc_ref[2] = 0.0

    t = t_ref[0]
    mcol = t != PADDING_IDX
    mf = mcol.astype(jnp.float32)
    xb = x_ref[...]
    colid = lax.broadcasted_iota(jnp.int32, (RB, CB), 1)
    acc_ref[0] += jnp.sum(xb * mf)
    acc_ref[1] += jnp.sum(jnp.where((colid == t) & mcol, xb, 0.0))
    acc_ref[2] += jnp.sum(mf)

    @pl.when(i == NI - 1)
    def _fin():
        oa_ref[0, 0] = acc_ref[0]
        ob_ref[0, 0] = acc_ref[1]
        ot_ref[0, 0] = acc_ref[2]


def _tc_part(x, target):
    t3 = target[:R_TC].reshape(NI, RB, 1)
    scalar_spec = pl.BlockSpec((1, 1), lambda i: (0, 0),
                               memory_space=pltpu.SMEM)
    outs = pl.pallas_call(
        _tc_body,
        grid=(NI,),
        in_specs=[
            pl.BlockSpec((1, RB, 1), lambda i: (i, 0, 0)),
            pl.BlockSpec((RB, CB), lambda i: (i, 0)),
        ],
        out_specs=[scalar_spec, scalar_spec, scalar_spec],
        out_shape=[jax.ShapeDtypeStruct((1, 1), jnp.float32)] * 3,
        scratch_shapes=[pltpu.SMEM((3,), jnp.float32)],
    )(t3, x)
    return outs[0][0, 0], outs[1][0, 0], outs[2][0, 0]


def _sc_kernel(x_hbm, texp_hbm, mexp_hbm, a_hbm, b_hbm, tok_hbm,
               texp_v, mexp_v, out_v, row0, row1, sem0, sem1):
    wid = lax.axis_index("s") * 2 + lax.axis_index("c")
    vzero = jnp.zeros((L,), jnp.float32)
    lanes = lax.iota(jnp.int32, L)
    dbase = R_TC + wid * BPWD
    pltpu.sync_copy(texp_hbm.at[pl.ds(wid * BPWD * L, BPWD * L)], texp_v)
    pltpu.sync_copy(mexp_hbm.at[pl.ds(wid * BPWD * L, BPWD * L)], mexp_v)

    UNROLL = 16

    def row_scan(buf, t16):
        tl = t16 - lanes

        def it(k, accs):
            a0, a1, a2, a3, g = accs
            o = k * (UNROLL * L)
            tlk = tl - o
            aa = [a0, a1, a2, a3]
            for j in range(UNROLL):
                c = buf[pl.ds(o + j * L, L)]
                aa[j % 4] = aa[j % 4] + c
                g = g + jnp.where(tlk == j * L, c, 0.0)
            return (aa[0], aa[1], aa[2], aa[3], g)
        a0, a1, a2, a3, g = lax.fori_loop(0, SIZE // (UNROLL * L), it,
                                          (vzero,) * 5)
        return (a0 + a1) + (a2 + a3), g

    bufs = (row0, row1)
    sems = (sem0, sem1)
    cps = [None, None]
    cps[0] = pltpu.async_copy(x_hbm.at[dbase], row0, sem0)
    aacc = vzero
    bacc = vzero
    tacc = vzero
    for r in range(BPWD):
        b = r % 2
        if r + 1 < BPWD:
            nb = (r + 1) % 2
            cps[nb] = pltpu.async_copy(
                x_hbm.at[dbase + r + 1], bufs[nb], sems[nb])
        cps[b].wait()
        t16 = texp_v[pl.ds(r * L, L)]
        mrow = mexp_v[pl.ds(r * L, L)]
        rowvec, g16 = row_scan(bufs[b], t16)
        aacc = aacc + rowvec * mrow
        bacc = bacc + g16 * mrow
        tacc = tacc + mrow
    inv_l = jnp.full((L,), 1.0 / L, jnp.float32)
    out_v[...] = aacc
    pltpu.sync_copy(out_v, a_hbm.at[wid])
    out_v[...] = bacc
    pltpu.sync_copy(out_v, b_hbm.at[wid])
    out_v[...] = tacc * inv_l
    pltpu.sync_copy(out_v, tok_hbm.at[wid])


@functools.cache
def _make_sc_call():
    return functools.partial(
        pl.kernel,
        mesh=plsc.VectorSubcoreMesh(core_axis_name="c", subcore_axis_name="s"),
        out_type=[
            jax.ShapeDtypeStruct((NW, L), jnp.float32),
            jax.ShapeDtypeStruct((NW, L), jnp.float32),
            jax.ShapeDtypeStruct((NW, L), jnp.float32),
        ],
        scratch_types=[
            pltpu.VMEM((BPWD * L,), jnp.int32),
            pltpu.VMEM((BPWD * L,), jnp.float32),
            pltpu.VMEM((L,), jnp.float32),
            pltpu.VMEM((SIZE,), jnp.float32),
            pltpu.VMEM((SIZE,), jnp.float32),
            pltpu.SemaphoreType.DMA,
            pltpu.SemaphoreType.DMA,
        ],
    )(_sc_kernel)


def kernel(x, target):
    target = target.astype(jnp.int32)
    tsc = target[R_TC:]
    texp = jnp.broadcast_to(tsc[:, None], (R_SC, L)).reshape(-1)
    mexp = jnp.broadcast_to(
        (tsc != PADDING_IDX).astype(jnp.float32)[:, None],
        (R_SC, L)).reshape(-1)
    a_parts, b_parts, tok_parts = _make_sc_call()(x, texp, mexp)
    a_tc, b_tc, tok_tc = _tc_part(x, target)
    a_sum = a_tc + jnp.sum(a_parts)
    b_sum = b_tc + jnp.sum(b_parts)
    tokens = tok_tc + jnp.sum(tok_parts)
    c32 = jnp.float32(C_CONST)
    return c32 - (jnp.float32(EPS) * a_sum
                  + jnp.float32(CONFIDENCE - EPS) * b_sum) / tokens
